# Initial kernel scaffold; baseline (speedup 1.0000x reference)
import jax
import jax.numpy as jnp
from jax.experimental import pallas as pl


def kernel(users, items, rated_items, items_nums, trusts, trusts_nums, user_rate_i_num, user_trust_u_num, P, Q, W, Y, B_u, B_i, global_bias):
    raise NotImplementedError("write your pallas kernel here")


# speedup vs baseline: 6.3681x; 6.3681x over previous
import functools

import jax
import jax.numpy as jnp
from jax import lax
from jax.experimental import pallas as pl
from jax.experimental.pallas import tpu as pltpu
from jax.experimental.pallas import tpu_sc as plsc

NC = 2
NS = 16
NW = NC * NS
L = 16


def _rsqrt_newton(x):
    i = lax.bitcast_convert_type(x, jnp.int32)
    i = jnp.int32(0x5F3759DF) - (i >> 1)
    y = lax.bitcast_convert_type(i, jnp.float32)
    for _ in range(3):
        y = y * (1.5 - 0.5 * x * y * y)
    return y


def _build_sc_kernel(B, D, HIST):
    CB = B // NW
    G = CB // L
    DC = D // L
    mesh = plsc.VectorSubcoreMesh(core_axis_name="c", subcore_axis_name="s")
    f32 = jnp.float32
    i32 = jnp.int32

    out_type = (
        jax.ShapeDtypeStruct((B,), f32),
        jax.ShapeDtypeStruct((B,), f32),
        jax.ShapeDtypeStruct((B,), f32),
        jax.ShapeDtypeStruct((B,), f32),
        jax.ShapeDtypeStruct((B,), f32),
        jax.ShapeDtypeStruct((B,), f32),
        jax.ShapeDtypeStruct((B, D), f32),
        jax.ShapeDtypeStruct((B, D), f32),
        jax.ShapeDtypeStruct((B,), f32),
        jax.ShapeDtypeStruct((B,), f32),
        jax.ShapeDtypeStruct((B, D), f32),
        jax.ShapeDtypeStruct((B, D), f32),
    )
    scratch_types = [
        pltpu.VMEM((CB,), i32),
        pltpu.VMEM((CB,), i32),
        pltpu.VMEM((HIST, CB), i32),
        pltpu.VMEM((HIST, CB), i32),
        pltpu.VMEM((CB,), f32),
        pltpu.VMEM((CB,), f32),
        pltpu.VMEM((CB,), f32),
        pltpu.VMEM((CB,), f32),
        pltpu.VMEM((L,), f32),
        pltpu.VMEM((CB, D), f32),
        pltpu.VMEM((CB, D), f32),
        pltpu.VMEM((CB, D), f32),
        pltpu.VMEM((CB, D), f32),
        pltpu.VMEM((CB, D), f32),
        pltpu.VMEM((CB,), f32),
        pltpu.VMEM((CB,), f32),
        pltpu.VMEM((CB, D), f32),
        pltpu.VMEM((CB, D), f32),
        pltpu.VMEM((CB, D), f32),
        pltpu.VMEM((CB, D), f32),
        pltpu.VMEM((CB, D), f32),
        pltpu.VMEM((CB, D), f32),
        pltpu.VMEM((CB,), f32),
        pltpu.VMEM((CB,), f32),
        pltpu.VMEM((CB,), f32),
        pltpu.VMEM((CB,), f32),
        pltpu.VMEM((CB,), f32),
        pltpu.VMEM((CB,), f32),
        pltpu.SemaphoreType.DMA,
        pltpu.SemaphoreType.DMA,
        pltpu.SemaphoreType.DMA,
        pltpu.SemaphoreType.DMA,
        pltpu.SemaphoreType.DMA,
    ]

    @functools.partial(
        pl.kernel, out_type=out_type, mesh=mesh,
        scratch_types=scratch_types,
        compiler_params=pltpu.CompilerParams(needs_layout_passes=False,
                                             use_tc_tiling_on_sc=False))
    def sc_kernel(users_h, items_h, tw_h, rw_h, ninv_h, ntnv_h, nuri_h,
                  nutu_h, P_h, Q_h, W_h, Y_h, bu_h, bi_h, gb_h,
                  rate_o, link_o, bu_o, bi_o, fiu_o, ftu_o, p_o, q_o,
                  fui_o, ftup_o, yi_o, wu_o,
                  uidx, iidx, twv, rwv, ninv, ntnv, nuri, nutu, gbv,
                  pbuf, qbuf, wubuf, yibuf, wvbuf, bub, bib, accw, accy,
                  stw0, stw1, sty0, sty1, fiu, ftu, fui, ftup, ratev,
                  linkv, semrows, semw0, semw1, semy0, semy1):
        wid = lax.axis_index("s") * NC + lax.axis_index("c")
        base = wid * CB
        bsl = pl.ds(base, CB)

        pltpu.sync_copy(users_h.at[bsl], uidx)
        pltpu.sync_copy(items_h.at[bsl], iidx)
        pltpu.sync_copy(tw_h.at[wid], twv)
        pltpu.sync_copy(rw_h.at[wid], rwv)
        pltpu.sync_copy(ninv_h.at[bsl], ninv)
        pltpu.sync_copy(ntnv_h.at[bsl], ntnv)
        pltpu.sync_copy(nuri_h.at[bsl], nuri)
        pltpu.sync_copy(nutu_h.at[bsl], nutu)
        pltpu.sync_copy(gb_h, gbv)

        drows = [
            pltpu.async_copy(P_h.at[uidx], pbuf, semrows),
            pltpu.async_copy(Q_h.at[iidx], qbuf, semrows),
            pltpu.async_copy(W_h.at[uidx], wubuf, semrows),
            pltpu.async_copy(Y_h.at[iidx], yibuf, semrows),
            pltpu.async_copy(W_h.at[twv.at[0]], wvbuf, semrows),
            pltpu.async_copy(bu_h.at[uidx], bub, semrows),
            pltpu.async_copy(bi_h.at[iidx], bib, semrows),
        ]

        zero = jnp.zeros((L,), f32)

        @pl.loop(0, CB)
        def _zero(i):
            for c in range(DC):
                accw[i, pl.ds(c * L, L)] = zero
                accy[i, pl.ds(c * L, L)] = zero

        def fire(j, stw, semw, sty, semy):
            pltpu.async_copy(W_h.at[twv.at[j]], stw, semw)
            pltpu.async_copy(Y_h.at[rwv.at[j]], sty, semy)

        def wait(buf, sem):
            pltpu.make_async_copy(W_h.at[pl.ds(0, CB)], buf, sem).wait()

        def accum(st, acc):
            @pl.loop(0, CB)
            def _(i):
                for c in range(DC):
                    sl = pl.ds(c * L, L)
                    plsc.addupdate(acc.at[i, sl], st[i, sl])

        fire(0, stw0, semw0, sty0, semy0)

        @pl.loop(0, HIST // 2)
        def _hist(h):
            j = h * 2
            fire(j + 1, stw1, semw1, sty1, semy1)
            wait(stw0, semw0)
            wait(sty0, semy0)
            accum(stw0, accw)
            accum(sty0, accy)

            @pl.when(j + 2 < HIST)
            def _():
                fire(j + 2, stw0, semw0, sty0, semy0)

            wait(stw1, semw1)
            wait(sty1, semy1)
            accum(stw1, accw)
            accum(sty1, accy)

        for d in drows:
            d.wait()

        pltpu.sync_copy(pbuf, p_o.at[bsl])
        pltpu.sync_copy(qbuf, q_o.at[bsl])
        pltpu.sync_copy(wubuf, wu_o.at[bsl])
        pltpu.sync_copy(yibuf, yi_o.at[bsl])
        pltpu.sync_copy(bub, bu_o.at[bsl])
        pltpu.sync_copy(bib, bi_o.at[bsl])

        eps = 1e-8
        gb = gbv[...]

        @pl.loop(0, G)
        def _grp(g):
            s = g * L
            ssl = pl.ds(s, L)
            rows = s + lax.iota(i32, L)
            iu = jnp.minimum(_rsqrt_newton(ninv[ssl] + eps), 1.0)
            tu = jnp.minimum(_rsqrt_newton(ntnv[ssl] + eps), 1.0)
            ui = jnp.minimum(_rsqrt_newton(nuri[ssl] + eps), 1.0)
            tup = jnp.minimum(_rsqrt_newton(nutu[ssl] + eps), 1.0)
            fiu[ssl] = iu
            ftu[ssl] = tu
            fui[ssl] = ui
            ftup[ssl] = tup

            def dbody(d, carry):
                ar, al = carry
                dv = lax.broadcast(d, (L,))
                pd = plsc.load_gather(pbuf, [rows, dv])
                qd = plsc.load_gather(qbuf, [rows, dv])
                wd = plsc.load_gather(accw, [rows, dv])
                yd = plsc.load_gather(accy, [rows, dv])
                wvd = plsc.load_gather(wvbuf, [rows, dv])
                ur = pd + iu * yd + tu * wd
                return (ar + qd * ur, al + wvd * pd)

            z = jnp.zeros((L,), f32)
            ar, al = lax.fori_loop(0, D, dbody, (z, z))
            ratev[ssl] = ar + bub[ssl] + bib[ssl] + gb
            linkv[ssl] = al

        pltpu.sync_copy(ratev, rate_o.at[bsl])
        pltpu.sync_copy(linkv, link_o.at[bsl])
        pltpu.sync_copy(fiu, fiu_o.at[bsl])
        pltpu.sync_copy(ftu, ftu_o.at[bsl])
        pltpu.sync_copy(fui, fui_o.at[bsl])
        pltpu.sync_copy(ftup, ftup_o.at[bsl])

    return sc_kernel


def kernel(users, items, rated_items, items_nums, trusts, trusts_nums,
           user_rate_i_num, user_trust_u_num, P, Q, W, Y, B_u, B_i,
           global_bias):
    B = users.shape[0]
    HIST = trusts.shape[1]
    D = P.shape[1]
    CB = B // NW

    i32 = jnp.int32
    tw = jnp.swapaxes(trusts.astype(i32).T.reshape(HIST, NW, CB), 0, 1)
    rw = jnp.swapaxes(rated_items.astype(i32).T.reshape(HIST, NW, CB), 0, 1)
    gb16 = jnp.full((L,), global_bias, jnp.float32)

    sc = _build_sc_kernel(B, D, HIST)
    (rate, link, bu, bi, fiu, ftu, p, q, fui, ftup, yi, wu) = sc(
        users.astype(i32), items.astype(i32), tw, rw,
        items_nums, trusts_nums, user_rate_i_num, user_trust_u_num,
        P, Q, W, Y, B_u.reshape(-1), B_i.reshape(-1), gb16)

    col = lambda v: v.reshape(B, 1)
    return (col(rate), link, col(bu), col(bi), col(fiu), col(ftu), p, q,
            col(fui), col(ftup), yi, wu)
